# Initial kernel scaffold; baseline (speedup 1.0000x reference)
import jax
import jax.numpy as jnp
from jax.experimental import pallas as pl


def kernel(inputs, embedding_weight):
    raise NotImplementedError("write your pallas kernel here")


# speedup vs baseline: 6.4924x; 6.4924x over previous
import functools

import jax
import jax.numpy as jnp
from jax import lax
from jax.experimental import pallas as pl
from jax.experimental.pallas import tpu as pltpu
from jax.experimental.pallas import tpu_sc as plsc

TOPK = 3
COMMIT = 0.25
BIG_ID = 2**30


def _dist_top3_body(nk, x_ref, w_ref, d_ref, idx_ref, loss_ref,
                    vals_scr, ids_scr, loss_scr):
    r = pl.program_id(0)
    k = pl.program_id(1)
    R = x_ref.shape[0]
    KB = w_ref.shape[0]

    x = x_ref[...]
    w = w_ref[...]
    xx = jnp.sum(x * x, axis=1, keepdims=True)
    ww = jnp.sum(w * w, axis=1)[None, :]
    xw = lax.dot_general(x, w, (((1,), (1,)), ((), ())),
                         preferred_element_type=jnp.float32)
    d = xx + ww - 2.0 * xw
    d_ref[...] = d

    @pl.when(k == 0)
    def _init():
        vals_scr[...] = jnp.full((R, 128), jnp.inf, jnp.float32)
        ids_scr[...] = jnp.full((R, 128), BIG_ID, jnp.int32)

    @pl.when(jnp.logical_and(r == 0, k == 0))
    def _init_loss():
        loss_scr[0] = 0.0

    lane_ids = (lax.broadcasted_iota(jnp.int32, (R, KB), 1)
                + k * KB)
    c = jnp.concatenate([d, vals_scr[...]], axis=1)
    cid = jnp.concatenate([lane_ids, ids_scr[...]], axis=1)

    mins = []
    sels = []
    for _ in range(TOPK):
        m = jnp.min(c, axis=1, keepdims=True)
        sel = jnp.min(jnp.where(c == m, cid, BIG_ID), axis=1, keepdims=True)
        mins.append(m)
        sels.append(sel)
        c = jnp.where(cid == sel, jnp.inf, c)

    lane = lax.broadcasted_iota(jnp.int32, (R, 128), 1)
    nv = jnp.full((R, 128), jnp.inf, jnp.float32)
    ni = jnp.full((R, 128), BIG_ID, jnp.int32)
    for t in range(TOPK - 1, -1, -1):
        nv = jnp.where(lane == t, mins[t], nv)
        ni = jnp.where(lane == t, sels[t], ni)
    vals_scr[...] = nv
    ids_scr[...] = ni

    @pl.when(k == nk - 1)
    def _emit():
        idx_ref[...] = ids_scr[...]
        top = jnp.where(lane < TOPK, vals_scr[...], 0.0)
        loss_scr[0] += jnp.sum(top)

    @pl.when(jnp.logical_and(r == pl.num_programs(0) - 1, k == nk - 1))
    def _emit_loss():
        n_total = R * pl.num_programs(0)
        v = COMMIT * loss_scr[0] / (n_total * TOPK * x_ref.shape[1])
        loss_ref[...] = jnp.reshape(v, (1, 1))


def _dist_top3(flat, emb, R=2048, KB=1024):
    N, C = flat.shape
    K = emb.shape[0]
    nr, nk = N // R, K // KB
    body = functools.partial(_dist_top3_body, nk)
    return pl.pallas_call(
        body,
        grid=(nr, nk),
        in_specs=[
            pl.BlockSpec((R, C), lambda r, k: (r, 0)),
            pl.BlockSpec((KB, C), lambda r, k: (k, 0)),
        ],
        out_specs=[
            pl.BlockSpec((R, KB), lambda r, k: (r, k)),
            pl.BlockSpec((R, 128), lambda r, k: (r, 0)),
            pl.BlockSpec((1, 1), lambda r, k: (0, 0)),
        ],
        out_shape=[
            jax.ShapeDtypeStruct((N, K), jnp.float32),
            jax.ShapeDtypeStruct((N, 128), jnp.int32),
            jax.ShapeDtypeStruct((1, 1), jnp.float32),
        ],
        scratch_shapes=[
            pltpu.VMEM((R, 128), jnp.float32),
            pltpu.VMEM((R, 128), jnp.int32),
            pltpu.SMEM((1,), jnp.float32),
        ],
    )(flat, emb)


def _gather_hist(emb, idx_flat, K):
    info = plsc.get_sparse_core_info()
    NC, NS, L = info.num_cores, info.num_subcores, info.num_lanes
    NW = NC * NS
    M = idx_flat.shape[0]
    C = emb.shape[1]
    per_w = M // NW
    CH = 128
    n_ch = per_w // CH
    ZR = K // NS
    mesh = plsc.VectorSubcoreMesh(core_axis_name="c", subcore_axis_name="s")

    @functools.partial(
        pl.kernel,
        out_type=[
            jax.ShapeDtypeStruct((M, C), jnp.float32),
            jax.ShapeDtypeStruct((NW, K), jnp.float32),
        ],
        mesh=mesh,
        scratch_types=[
            pltpu.VMEM((CH,), jnp.int32),
            pltpu.VMEM((CH, C), jnp.float32),
            pltpu.VMEM((K,), jnp.float32),
            pltpu.SemaphoreType.DMA,
        ],
        compiler_params=pltpu.CompilerParams(needs_layout_passes=False),
    )
    def k(emb_hbm, idx_hbm, outq_hbm, cnt_hbm, idx_v, rows_v, cnt_v, sem):
        cid = lax.axis_index("c")
        sid = lax.axis_index("s")
        wid = sid * NC + cid
        base = wid * per_w

        def zero_body(i, _):
            cnt_v[pl.ds(i * L, L)] = jnp.zeros((L,), jnp.float32)
            return 0
        lax.fori_loop(0, K // L, zero_body, 0)

        ones = jnp.ones((L,), jnp.float32)
        for c in range(n_ch):
            pltpu.sync_copy(idx_hbm.at[pl.ds(base + c * CH, CH)], idx_v)
            pltpu.async_copy(emb_hbm.at[idx_v], rows_v, sem).wait()
            pltpu.sync_copy(rows_v, outq_hbm.at[pl.ds(base + c * CH, CH)])
            for j in range(CH // L):
                v = idx_v[pl.ds(j * L, L)]
                plsc.addupdate_scatter(cnt_v, [v], ones)

        pltpu.sync_copy(cnt_v, cnt_hbm.at[wid])

    return k(emb, idx_flat)


def _finalize_body(n_total, cnt_ref, avg_ref, perp_ref):
    counts = jnp.sum(cnt_ref[...], axis=0, keepdims=True)
    avg = counts / n_total
    avg_ref[...] = avg
    ent = jnp.sum(avg * jnp.log(avg + 1e-10))
    perp_ref[...] = jnp.reshape(jnp.exp(-ent), (1, 1))


def _finalize(cnt, n_total):
    NW, K = cnt.shape
    return pl.pallas_call(
        functools.partial(_finalize_body, n_total),
        out_specs=[
            pl.BlockSpec((1, K), lambda: (0, 0)),
            pl.BlockSpec((1, 1), lambda: (0, 0)),
        ],
        out_shape=[
            jax.ShapeDtypeStruct((1, K), jnp.float32),
            jax.ShapeDtypeStruct((1, 1), jnp.float32),
        ],
    )(cnt)


def kernel(inputs, embedding_weight):
    B, T, C = inputs.shape
    K = embedding_weight.shape[0]
    N = B * T
    flat = inputs.reshape(N, C)

    distances, idx_pad, loss = _dist_top3(flat, embedding_weight)
    idx3 = idx_pad[:, :TOPK]
    idx_flat = idx3.reshape(-1)

    quantized_flat, cnt = _gather_hist(embedding_weight, idx_flat, K)
    avg, perp = _finalize(cnt, N)

    loss_out = loss.reshape(())
    quantized_st = quantized_flat.reshape(B, T, TOPK, C)
    perplexity = perp.reshape(())
    avg_probs = avg.reshape(K)
    encoding_indices_out = idx3.reshape(B, T, TOPK)
    distances_out = distances.reshape(B, T, K)
    return (loss_out, quantized_st, perplexity, avg_probs,
            encoding_indices_out, distances_out)
